# Initial kernel scaffold; baseline (speedup 1.0000x reference)
import jax
import jax.numpy as jnp
from jax.experimental import pallas as pl


def kernel(logits, context, target, context_maps, context_bias, weights):
    raise NotImplementedError("write your pallas kernel here")


# speedup vs baseline: 2.5765x; 2.5765x over previous
import jax
import jax.numpy as jnp
from jax.experimental import pallas as pl

_CLASSES = 10
_SIZE = 64
_CMS = 8
_CTX = 128
_IN = 512
_B = 128
_ROWS = 2 ** _CMS
_LR = 0.01
_PRED_CLIP = 0.01
_WEIGHT_CLIP = 5.0
_CS = _CLASSES * _SIZE


def _logit(x):
    return jnp.log(x) - jnp.log1p(-x)


def _body(maps_ref, bias_ref, ctxT_ref, lg_ref, lgTh_ref, lgTl_ref, tgt_ref,
          w_ref, out_ref, nw_ref):
    f32 = jnp.float32
    d = jnp.dot(maps_ref[0, 0], ctxT_ref[...],
                preferred_element_type=f32)
    bits = (d > bias_ref[0]).astype(jnp.int32)
    pow2 = jnp.left_shift(
        1, jax.lax.broadcasted_iota(jnp.int32, (_CMS, _B), 0))
    idx = jnp.sum(bits * pow2, axis=0, keepdims=True)

    w = w_ref[0, 0]
    p = jnp.dot(w, lg_ref[0], preferred_element_type=f32)

    jiota = jax.lax.broadcasted_iota(jnp.int32, (_ROWS, _B), 0)
    onehot_t = jiota == idx
    out = jnp.sum(jnp.where(onehot_t, p, 0.0), axis=0,
                  keepdims=True)
    lo = _logit(f32(_PRED_CLIP))
    hi = _logit(f32(1.0 - _PRED_CLIP))
    outc = jnp.clip(out, lo, hi)
    out_ref[0, 0] = outc
    diff = jax.nn.sigmoid(outc) - tgt_ref[0]

    biota = jax.lax.broadcasted_iota(jnp.int32, (_ROWS, _B), 1)
    wins = jnp.max(jnp.where(onehot_t, biota, -1),
                   axis=1, keepdims=True)
    ld = _LR * diff
    ld_hi = ld.astype(jnp.bfloat16)
    ld_lo = (ld - ld_hi.astype(f32)).astype(jnp.bfloat16)
    winner = biota == wins
    wsel_hi = jnp.where(winner, ld_hi.astype(f32),
                        0.0).astype(jnp.bfloat16)
    wsel_lo = jnp.where(winner, ld_lo.astype(f32),
                        0.0).astype(jnp.bfloat16)
    lgT_hi = lgTh_ref[0]
    lgT_lo = lgTl_ref[0]
    upd = (jnp.dot(wsel_hi, lgT_hi, preferred_element_type=f32)
           + jnp.dot(wsel_hi, lgT_lo, preferred_element_type=f32)
           + jnp.dot(wsel_lo, lgT_hi, preferred_element_type=f32))
    nw_ref[0, 0] = jnp.where(
        wins >= 0, jnp.clip(w - upd, -_WEIGHT_CLIP, _WEIGHT_CLIP), w)


@jax.jit
def kernel(logits, context, target, context_maps, context_bias, weights):
    f32 = jnp.float32
    ctxT = context.T
    bias = context_bias.reshape(_CS, _CMS, 1)
    lgT = jnp.transpose(logits, (0, 2, 1))
    lgT_hi = lgT.astype(jnp.bfloat16)
    lgT_lo = (lgT - lgT_hi.astype(f32)).astype(jnp.bfloat16)
    tgt = target.reshape(_CLASSES, 1, _B)

    out_logits, new_weights = pl.pallas_call(
        _body,
        grid=(_CLASSES, _SIZE),
        in_specs=[
            pl.BlockSpec((1, 1, _CMS, _CTX), lambda c, s: (c, s, 0, 0)),
            pl.BlockSpec((1, _CMS, 1), lambda c, s: (c * _SIZE + s, 0, 0)),
            pl.BlockSpec((_CTX, _B), lambda c, s: (0, 0)),
            pl.BlockSpec((1, _IN, _B), lambda c, s: (c, 0, 0)),
            pl.BlockSpec((1, _B, _IN), lambda c, s: (c, 0, 0)),
            pl.BlockSpec((1, _B, _IN), lambda c, s: (c, 0, 0)),
            pl.BlockSpec((1, 1, _B), lambda c, s: (c, 0, 0)),
            pl.BlockSpec((1, 1, _ROWS, _IN), lambda c, s: (c, s, 0, 0)),
        ],
        out_specs=[
            pl.BlockSpec((1, 1, 1, _B), lambda c, s: (c, s, 0, 0)),
            pl.BlockSpec((1, 1, _ROWS, _IN), lambda c, s: (c, s, 0, 0)),
        ],
        out_shape=[
            jax.ShapeDtypeStruct((_CLASSES, _SIZE, 1, _B), f32),
            jax.ShapeDtypeStruct((_CLASSES, _SIZE, _ROWS, _IN), f32),
        ],
    )(context_maps, bias, ctxT, logits, lgT_hi, lgT_lo, tgt, weights)
    return out_logits.reshape(_CLASSES, _SIZE, _B), new_weights


# speedup vs baseline: 5.0730x; 1.9689x over previous
import jax
import jax.numpy as jnp
from jax.experimental import pallas as pl

_CLASSES = 10
_SIZE = 64
_CMS = 8
_CTX = 128
_IN = 512
_B = 128
_ROWS = 2 ** _CMS
_LR = 0.01
_PRED_CLIP = 0.01
_WEIGHT_CLIP = 5.0
_CS = _CLASSES * _SIZE
_SB = 4


def _logit(x):
    return jnp.log(x) - jnp.log1p(-x)


def _body(maps_ref, bias_ref, ctxT_ref, lg_ref, lgTh_ref, lgTl_ref, tgt_ref,
          w_ref, out_ref, nw_ref):
    f32 = jnp.float32
    lo = _logit(f32(_PRED_CLIP))
    hi = _logit(f32(1.0 - _PRED_CLIP))
    ctxT = ctxT_ref[...]
    lg = lg_ref[0]
    lgT_hi = lgTh_ref[0]
    lgT_lo = lgTl_ref[0]
    tgt = tgt_ref[0]
    for si in range(_SB):
        d = jnp.dot(maps_ref[0, si], ctxT,
                    preferred_element_type=f32)
        bits = (d > bias_ref[0, si]).astype(jnp.int32)
        pow2 = jnp.left_shift(
            1, jax.lax.broadcasted_iota(jnp.int32, (_CMS, _B), 0))
        idx = jnp.sum(bits * pow2, axis=0, keepdims=True)

        w = w_ref[0, si]
        p = jnp.dot(w, lg, preferred_element_type=f32)

        jiota = jax.lax.broadcasted_iota(jnp.int32, (_ROWS, _B), 0)
        onehot_t = jiota == idx
        out = jnp.sum(jnp.where(onehot_t, p, 0.0), axis=0,
                      keepdims=True)
        outc = jnp.clip(out, lo, hi)
        out_ref[0, si] = outc
        diff = jax.nn.sigmoid(outc) - tgt

        biota = jax.lax.broadcasted_iota(jnp.int32, (_ROWS, _B), 1)
        wins = jnp.max(jnp.where(onehot_t, biota, -1),
                       axis=1, keepdims=True)
        ld = _LR * diff
        ld_hi = ld.astype(jnp.bfloat16)
        ld_lo = (ld - ld_hi.astype(f32)).astype(jnp.bfloat16)
        winner = biota == wins
        wsel_hi = jnp.where(winner, ld_hi.astype(f32),
                            0.0).astype(jnp.bfloat16)
        wsel_lo = jnp.where(winner, ld_lo.astype(f32),
                            0.0).astype(jnp.bfloat16)
        upd = (jnp.dot(wsel_hi, lgT_hi, preferred_element_type=f32)
               + jnp.dot(wsel_hi, lgT_lo, preferred_element_type=f32)
               + jnp.dot(wsel_lo, lgT_hi, preferred_element_type=f32))
        nw_ref[0, si] = jnp.where(
            wins >= 0, jnp.clip(w - upd, -_WEIGHT_CLIP, _WEIGHT_CLIP), w)


@jax.jit
def kernel(logits, context, target, context_maps, context_bias, weights):
    f32 = jnp.float32
    ctxT = context.T
    bias = context_bias.reshape(_CLASSES, _SIZE, _CMS, 1)
    lgT = jnp.transpose(logits, (0, 2, 1))
    lgT_hi = lgT.astype(jnp.bfloat16)
    lgT_lo = (lgT - lgT_hi.astype(f32)).astype(jnp.bfloat16)
    tgt = target.reshape(_CLASSES, 1, _B)

    out_logits, new_weights = pl.pallas_call(
        _body,
        grid=(_CLASSES, _SIZE // _SB),
        in_specs=[
            pl.BlockSpec((1, _SB, _CMS, _CTX), lambda c, s: (c, s, 0, 0)),
            pl.BlockSpec((1, _SB, _CMS, 1), lambda c, s: (c, s, 0, 0)),
            pl.BlockSpec((_CTX, _B), lambda c, s: (0, 0)),
            pl.BlockSpec((1, _IN, _B), lambda c, s: (c, 0, 0)),
            pl.BlockSpec((1, _B, _IN), lambda c, s: (c, 0, 0)),
            pl.BlockSpec((1, _B, _IN), lambda c, s: (c, 0, 0)),
            pl.BlockSpec((1, 1, _B), lambda c, s: (c, 0, 0)),
            pl.BlockSpec((1, _SB, _ROWS, _IN), lambda c, s: (c, s, 0, 0)),
        ],
        out_specs=[
            pl.BlockSpec((1, _SB, 1, _B), lambda c, s: (c, s, 0, 0)),
            pl.BlockSpec((1, _SB, _ROWS, _IN), lambda c, s: (c, s, 0, 0)),
        ],
        out_shape=[
            jax.ShapeDtypeStruct((_CLASSES, _SIZE, 1, _B), f32),
            jax.ShapeDtypeStruct((_CLASSES, _SIZE, _ROWS, _IN), f32),
        ],
    )(context_maps, bias, ctxT, logits, lgT_hi, lgT_lo, tgt, weights)
    return out_logits.reshape(_CLASSES, _SIZE, _B), new_weights


# speedup vs baseline: 5.8276x; 1.1488x over previous
import jax
import jax.numpy as jnp
from jax.experimental import pallas as pl

_CLASSES = 10
_SIZE = 64
_CMS = 8
_CTX = 128
_IN = 512
_B = 128
_ROWS = 2 ** _CMS
_LR = 0.01
_PRED_CLIP = 0.01
_WEIGHT_CLIP = 5.0
_CS = _CLASSES * _SIZE
_SB = 8


def _logit(x):
    return jnp.log(x) - jnp.log1p(-x)


def _body(maps_ref, bias_ref, ctxT_ref, lg_ref, lgTh_ref, lgTl_ref, tgt_ref,
          w_ref, out_ref, nw_ref):
    f32 = jnp.float32
    lo = _logit(f32(_PRED_CLIP))
    hi = _logit(f32(1.0 - _PRED_CLIP))
    ctxT = ctxT_ref[...]
    lg = lg_ref[0]
    lgT_hi = lgTh_ref[0]
    lgT_lo = lgTl_ref[0]
    tgt = tgt_ref[0]
    for si in range(_SB):
        d = jnp.dot(maps_ref[0, si], ctxT,
                    preferred_element_type=f32)
        bits = (d > bias_ref[0, si]).astype(jnp.int32)
        pow2 = jnp.left_shift(
            1, jax.lax.broadcasted_iota(jnp.int32, (_CMS, _B), 0))
        idx = jnp.sum(bits * pow2, axis=0, keepdims=True)

        w = w_ref[0, si]
        p = jnp.dot(w, lg, preferred_element_type=f32)

        jiota = jax.lax.broadcasted_iota(jnp.int32, (_ROWS, _B), 0)
        onehot_t = jiota == idx
        out = jnp.sum(jnp.where(onehot_t, p, 0.0), axis=0,
                      keepdims=True)
        outc = jnp.clip(out, lo, hi)
        out_ref[0, si] = outc
        diff = jax.nn.sigmoid(outc) - tgt

        biota = jax.lax.broadcasted_iota(jnp.int32, (_ROWS, _B), 1)
        wins = jnp.max(jnp.where(onehot_t, biota, -1),
                       axis=1, keepdims=True)
        ld = _LR * diff
        ld_hi = ld.astype(jnp.bfloat16)
        ld_lo = (ld - ld_hi.astype(f32)).astype(jnp.bfloat16)
        winner = biota == wins
        wsel_hi = jnp.where(winner, ld_hi.astype(f32),
                            0.0).astype(jnp.bfloat16)
        wsel_lo = jnp.where(winner, ld_lo.astype(f32),
                            0.0).astype(jnp.bfloat16)
        upd = (jnp.dot(wsel_hi, lgT_hi, preferred_element_type=f32)
               + jnp.dot(wsel_hi, lgT_lo, preferred_element_type=f32)
               + jnp.dot(wsel_lo, lgT_hi, preferred_element_type=f32))
        nw_ref[0, si] = jnp.where(
            wins >= 0, jnp.clip(w - upd, -_WEIGHT_CLIP, _WEIGHT_CLIP), w)


@jax.jit
def kernel(logits, context, target, context_maps, context_bias, weights):
    f32 = jnp.float32
    ctxT = context.T
    bias = context_bias.reshape(_CLASSES, _SIZE, _CMS, 1)
    lgT = jnp.transpose(logits, (0, 2, 1))
    lgT_hi = lgT.astype(jnp.bfloat16)
    lgT_lo = (lgT - lgT_hi.astype(f32)).astype(jnp.bfloat16)
    tgt = target.reshape(_CLASSES, 1, _B)

    out_logits, new_weights = pl.pallas_call(
        _body,
        grid=(_CLASSES, _SIZE // _SB),
        in_specs=[
            pl.BlockSpec((1, _SB, _CMS, _CTX), lambda c, s: (c, s, 0, 0)),
            pl.BlockSpec((1, _SB, _CMS, 1), lambda c, s: (c, s, 0, 0)),
            pl.BlockSpec((_CTX, _B), lambda c, s: (0, 0)),
            pl.BlockSpec((1, _IN, _B), lambda c, s: (c, 0, 0)),
            pl.BlockSpec((1, _B, _IN), lambda c, s: (c, 0, 0)),
            pl.BlockSpec((1, _B, _IN), lambda c, s: (c, 0, 0)),
            pl.BlockSpec((1, 1, _B), lambda c, s: (c, 0, 0)),
            pl.BlockSpec((1, _SB, _ROWS, _IN), lambda c, s: (c, s, 0, 0)),
        ],
        out_specs=[
            pl.BlockSpec((1, _SB, 1, _B), lambda c, s: (c, s, 0, 0)),
            pl.BlockSpec((1, _SB, _ROWS, _IN), lambda c, s: (c, s, 0, 0)),
        ],
        out_shape=[
            jax.ShapeDtypeStruct((_CLASSES, _SIZE, 1, _B), f32),
            jax.ShapeDtypeStruct((_CLASSES, _SIZE, _ROWS, _IN), f32),
        ],
    )(context_maps, bias, ctxT, logits, lgT_hi, lgT_lo, tgt, weights)
    return out_logits.reshape(_CLASSES, _SIZE, _B), new_weights
